# Initial kernel scaffold; baseline (speedup 1.0000x reference)
import jax
import jax.numpy as jnp
from jax.experimental import pallas as pl


def kernel(sources):
    raise NotImplementedError("write your pallas kernel here")


# speedup vs baseline: 3.3217x; 3.3217x over previous
import jax
import jax.numpy as jnp
from jax.experimental import pallas as pl
from jax.experimental.pallas import tpu as pltpu


def _copy_body(gidx_ref, in_ref, out_ref):
    out_ref[...] = in_ref[...]


def kernel(sources):
    S, B, C, T = sources.shape
    flat = sources.reshape(S * B, C, T)

    perm = jnp.argsort(jax.random.uniform(jax.random.key(42), (B,)))
    gidx = jnp.concatenate(
        [perm.astype(jnp.int32), (B + jnp.arange(B)).astype(jnp.int32)]
    )

    out = pl.pallas_call(
        _copy_body,
        grid_spec=pltpu.PrefetchScalarGridSpec(
            num_scalar_prefetch=1,
            grid=(S * B,),
            in_specs=[
                pl.BlockSpec((1, C, T), lambda i, gidx_ref: (gidx_ref[i], 0, 0))
            ],
            out_specs=pl.BlockSpec((1, C, T), lambda i, gidx_ref: (i, 0, 0)),
        ),
        out_shape=jax.ShapeDtypeStruct((S * B, C, T), sources.dtype),
    )(gidx, flat)
    return out.reshape(S, B, C, T)
